# Initial kernel scaffold; baseline (speedup 1.0000x reference)
import jax
import jax.numpy as jnp
from jax.experimental import pallas as pl


def kernel(batch, table):
    raise NotImplementedError("write your pallas kernel here")


# speedup vs baseline: 2.6311x; 2.6311x over previous
import functools

import jax
import jax.numpy as jnp
from jax import lax
from jax.experimental import pallas as pl
from jax.experimental.pallas import tpu as pltpu
from jax.experimental.pallas import tpu_sc as plsc

BATCH = 1024
MAX_LEN = 200
D = 128

_info = plsc.get_sparse_core_info()
_NC, _NS = _info.num_cores, _info.num_subcores
NW = _NC * _NS
ROWS = BATCH * MAX_LEN
RPW = ROWS // NW
SENT_PW = RPW // MAX_LEN
CH = MAX_LEN // 2
CPW = RPW // CH


def _pe_body(o_ref):
    pos = lax.broadcasted_iota(jnp.int32, (MAX_LEN, D), 0).astype(jnp.float32)
    di = lax.broadcasted_iota(jnp.int32, (MAX_LEN, D), 1)
    deven = ((di // 2) * 2).astype(jnp.float32)
    ang = pos * jnp.exp(-(jnp.log(10000.0) / D) * deven)
    o_ref[...] = jnp.where(di % 2 == 0, jnp.sin(ang), jnp.cos(ang))


_pe_call = pl.pallas_call(
    _pe_body, out_shape=jax.ShapeDtypeStruct((MAX_LEN, D), jnp.float32))


_mesh = plsc.VectorSubcoreMesh(core_axis_name="c", subcore_axis_name="s")


@functools.partial(
    pl.kernel,
    mesh=_mesh,
    out_type=jax.ShapeDtypeStruct((ROWS, D), jnp.float32),
    scratch_types=[
        pltpu.VMEM((CPW, CH), jnp.int32),
        pltpu.VMEM((MAX_LEN, D), jnp.float32),
        pltpu.VMEM((MAX_LEN, D), jnp.float32),
        pltpu.SemaphoreType.DMA,
    ],
)
def _sc_embed(idx_hbm, table_hbm, pe_hbm, out_hbm, idx_v, pe_v, buf, sem):
    wid = lax.axis_index("s") * _NC + lax.axis_index("c")
    pltpu.sync_copy(idx_hbm.at[pl.ds(wid * CPW, CPW)], idx_v)
    pltpu.sync_copy(pe_hbm, pe_v)

    def sentence(s, carry):
        base = wid * RPW + s * MAX_LEN
        c1 = pltpu.async_copy(
            table_hbm.at[idx_v.at[2 * s]], buf.at[pl.ds(0, CH)], sem)
        c2 = pltpu.async_copy(
            table_hbm.at[idx_v.at[2 * s + 1]], buf.at[pl.ds(CH, CH)], sem)
        c1.wait()
        c2.wait()

        def row(r, rc):
            for c in range(D // 16):
                sl = pl.ds(c * 16, 16)
                plsc.addupdate(buf.at[r, sl], pe_v[r, sl])
            return rc

        lax.fori_loop(0, MAX_LEN, row, 0)
        pltpu.sync_copy(buf, out_hbm.at[pl.ds(base, MAX_LEN)])
        return carry

    lax.fori_loop(0, SENT_PW, sentence, 0)


def kernel(batch, table):
    pe = _pe_call()
    idx = batch.astype(jnp.int32).reshape(ROWS // CH, CH)
    out = _sc_embed(idx, table, pe)
    return out.reshape(BATCH, MAX_LEN, D)


# speedup vs baseline: 2.6670x; 1.0136x over previous
import functools

import jax
import jax.numpy as jnp
from jax import lax
from jax.experimental import pallas as pl
from jax.experimental.pallas import tpu as pltpu
from jax.experimental.pallas import tpu_sc as plsc

BATCH = 1024
MAX_LEN = 200
D = 128

_info = plsc.get_sparse_core_info()
_NC, _NS = _info.num_cores, _info.num_subcores
NW = _NC * _NS
ROWS = BATCH * MAX_LEN
RPW = ROWS // NW
SPW = RPW // MAX_LEN
CH = MAX_LEN // 2
NBUF = 3


def _pe_body(o_ref):
    pos = lax.broadcasted_iota(jnp.int32, (MAX_LEN, D), 0).astype(jnp.float32)
    di = lax.broadcasted_iota(jnp.int32, (MAX_LEN, D), 1)
    deven = ((di // 2) * 2).astype(jnp.float32)
    ang = pos * jnp.exp(-(jnp.log(10000.0) / D) * deven)
    o_ref[...] = jnp.where(di % 2 == 0, jnp.sin(ang), jnp.cos(ang))


_pe_call = pl.pallas_call(
    _pe_body, out_shape=jax.ShapeDtypeStruct((MAX_LEN, D), jnp.float32))


_mesh = plsc.VectorSubcoreMesh(core_axis_name="c", subcore_axis_name="s")


@functools.partial(
    pl.kernel,
    mesh=_mesh,
    out_type=jax.ShapeDtypeStruct((ROWS, D), jnp.float32),
    scratch_types=[
        pltpu.VMEM((2 * SPW, CH), jnp.int32),
        pltpu.VMEM((MAX_LEN, D), jnp.float32),
        pltpu.VMEM((NBUF, MAX_LEN, D), jnp.float32),
    ] + [pltpu.SemaphoreType.DMA] * (2 * NBUF),
)
def _sc_embed(idx_hbm, table_hbm, pe_hbm, out_hbm, idx_v, pe_v, bufs, *sems):
    gsem = sems[:NBUF]
    ssem = sems[NBUF:]
    wid = lax.axis_index("s") * _NC + lax.axis_index("c")
    pltpu.sync_copy(idx_hbm.at[pl.ds(wid * 2 * SPW, 2 * SPW)], idx_v)
    pltpu.sync_copy(pe_hbm, pe_v)
    wbase = wid * RPW

    def start_gathers(s, b):
        pltpu.async_copy(
            table_hbm.at[idx_v.at[2 * s]], bufs.at[b, pl.ds(0, CH)], gsem[b])
        pltpu.async_copy(
            table_hbm.at[idx_v.at[2 * s + 1]], bufs.at[b, pl.ds(CH, CH)],
            gsem[b])

    def wait_gathers(s, b):
        pltpu.make_async_copy(
            table_hbm.at[idx_v.at[2 * s]], bufs.at[b, pl.ds(0, CH)],
            gsem[b]).wait()
        pltpu.make_async_copy(
            table_hbm.at[idx_v.at[2 * s + 1]], bufs.at[b, pl.ds(CH, CH)],
            gsem[b]).wait()

    def start_scatter(s, b):
        pltpu.async_copy(
            bufs.at[b], out_hbm.at[pl.ds(wbase + s * MAX_LEN, MAX_LEN)],
            ssem[b])

    def wait_scatter(b):
        pltpu.make_async_copy(
            bufs.at[b], out_hbm.at[pl.ds(0, MAX_LEN)], ssem[b]).wait()

    def add_pe(b):
        def row(r, rc):
            for u in range(2):
                for c in range(D // 16):
                    sl = pl.ds(c * 16, 16)
                    plsc.addupdate(
                        bufs.at[b, 2 * r + u, sl], pe_v[2 * r + u, sl])
            return rc

        lax.fori_loop(0, MAX_LEN // 2, row, 0)

    def slot(s, b, wait_prev, next_s):
        bn = (b + 1) % NBUF
        if wait_prev:
            wait_scatter(bn)
        if next_s is not None:
            start_gathers(next_s, bn)
        wait_gathers(s, b)
        add_pe(b)
        start_scatter(s, b)

    start_gathers(0, 0)
    slot(0, 0, False, 1)
    slot(1, 1, False, 2)
    slot(2, 2, True, 3)

    def body(g, carry):
        s0 = 3 * g + 3
        for b in range(NBUF):
            slot(s0 + b, b, True, s0 + b + 1)
        return carry

    lax.fori_loop(0, (SPW - 5) // 3, body, 0)

    slot(SPW - 2, (SPW - 2) % NBUF, True, SPW - 1)
    slot(SPW - 1, (SPW - 1) % NBUF, True, None)
    wait_scatter((SPW - 2) % NBUF)
    wait_scatter((SPW - 1) % NBUF)


def kernel(batch, table):
    pe = _pe_call()
    idx = batch.astype(jnp.int32).reshape(2 * BATCH * MAX_LEN // MAX_LEN, CH)
    out = _sc_embed(idx, table, pe)
    return out.reshape(BATCH, MAX_LEN, D)
